# baseline (speedup 1.0000x reference)
import functools

import jax
import jax.numpy as jnp
from jax import lax
from jax.experimental import pallas as pl
from jax.experimental.pallas import tpu as pltpu
from jax.experimental.pallas import tpu_sc as plsc

N = 10000
D = 128
H = 128
DEG = 32

NC = 2
NS = 16
NW = NC * NS

CHUNK_D = 4
CHUNK_E = CHUNK_D * DEG
TRIPS = 80
NP = NW * TRIPS * CHUNK_D


def _mm_body(x_ref, w1a_ref, w1b_ref, b1_ref, a_ref, b_ref):
    x = x_ref[...]
    a_ref[...] = (
        jnp.dot(x, w1a_ref[...], preferred_element_type=jnp.float32)
        + b1_ref[...]
    )
    b_ref[...] = jnp.dot(x, w1b_ref[...], preferred_element_type=jnp.float32)


def _node_transform(x, w1a, w1b, b1):
    blk = 2000
    return pl.pallas_call(
        _mm_body,
        grid=(N // blk,),
        in_specs=[
            pl.BlockSpec((blk, D), lambda i: (i, 0)),
            pl.BlockSpec((D, H), lambda i: (0, 0)),
            pl.BlockSpec((D, H), lambda i: (0, 0)),
            pl.BlockSpec((1, H), lambda i: (0, 0)),
        ],
        out_specs=[
            pl.BlockSpec((blk, H), lambda i: (i, 0)),
            pl.BlockSpec((blk, H), lambda i: (i, 0)),
        ],
        out_shape=[
            jax.ShapeDtypeStruct((N, H), jnp.float32),
            jax.ShapeDtypeStruct((N, H), jnp.float32),
        ],
    )(x, w1a, w1b, b1)


_DNUMS = lax.GatherDimensionNumbers(
    offset_dims=(), collapsed_slice_dims=(0,), start_index_map=(0,)
)


def _perm(v, idx):
    return lax.gather(
        v, idx[:, None], _DNUMS, (1,),
        mode=lax.GatherScatterMode.PROMISE_IN_BOUNDS,
    )


def _edge_body(a_hbm, b_hbm, src_hbm, w2_hbm, b2_hbm, out_hbm,
               idx_v, rows_v, a_v, out_v, w2_v, b2_v, sem0, sem1):
    cid = lax.axis_index("c")
    sid = lax.axis_index("s")
    wid = sid * NC + cid

    pltpu.sync_copy(w2_hbm, w2_v)
    pltpu.sync_copy(b2_hbm, b2_v)

    b2vec = b2_v[...]
    iota16 = lax.iota(jnp.int32, 16)
    w_chunks = [w2_v[pl.ds(16 * jb, 16)] for jb in range(H // 16)]
    strides = (1, 2, 4, 8)
    perm_idx = [iota16 ^ s for s in strides]
    masks = [(iota16 & s) == 0 for s in strides]
    sems = (sem0, sem1)

    def base_of(t):
        return (t * NW + wid) * CHUNK_D

    def load_meta(t, p):
        base_d = base_of(t)
        pltpu.sync_copy(src_hbm.at[pl.ds(base_d * DEG, CHUNK_E)],
                        idx_v.at[p])
        pltpu.sync_copy(a_hbm.at[pl.ds(base_d, CHUNK_D)], a_v.at[p])

    def gather_start(p):
        pltpu.async_copy(b_hbm.at[idx_v.at[p]], rows_v.at[p], sems[p])

    def gather_wait(p):
        pltpu.make_async_copy(
            b_hbm.at[idx_v.at[p]], rows_v.at[p], sems[p]
        ).wait()

    def compute(t, p):
        def dst_body(d, _):
            a_chunks = [
                a_v[p, d, pl.ds(16 * jb, 16)] for jb in range(H // 16)
            ]
            for half in range(DEG // 16):
                level = []
                for e in range(16):
                    row = d * DEG + half * 16 + e
                    acc = None
                    for jb in range(H // 16):
                        g = rows_v[p, row, pl.ds(16 * jb, 16)]
                        t_ = g + a_chunks[jb]
                        u = jnp.maximum(t_, t_ * 0.01)
                        m = u * w_chunks[jb]
                        acc = m if acc is None else acc + m
                    level.append(acc)
                for li in range(4):
                    nxt = []
                    for q in range(0, len(level), 2):
                        va, vb = level[q], level[q + 1]
                        hi = jnp.where(masks[li], va, vb)
                        lo = jnp.where(masks[li], vb, va)
                        nxt.append(hi + _perm(lo, perm_idx[li]))
                    level = nxt
                out_v[d, pl.ds(half * 16, 16)] = level[0] + b2vec
            return 0

        for d in range(CHUNK_D):
            dst_body(d, 0)
        pltpu.sync_copy(out_v, out_hbm.at[pl.ds(base_of(t), CHUNK_D)])

    def chunk_body(t, _):
        load_meta(t, 0)
        gather_start(0)
        gather_wait(0)
        compute(t, 0)
        return 0

    lax.fori_loop(0, TRIPS, chunk_body, 0)


_edge_call = functools.partial(
    pl.kernel,
    mesh=plsc.VectorSubcoreMesh(core_axis_name="c", subcore_axis_name="s"),
    out_type=jax.ShapeDtypeStruct((NP, DEG), jnp.float32),
    scratch_types=[
        pltpu.VMEM((2, CHUNK_E), jnp.int32),
        pltpu.VMEM((2, CHUNK_E, H), jnp.float32),
        pltpu.VMEM((2, CHUNK_D, H), jnp.float32),
        pltpu.VMEM((CHUNK_D, DEG), jnp.float32),
        pltpu.VMEM((H,), jnp.float32),
        pltpu.VMEM((16,), jnp.float32),
        pltpu.SemaphoreType.DMA,
        pltpu.SemaphoreType.DMA,
    ],
)(_edge_body)


def kernel(node_feature, edge_index, W1, b1, W2, b2):
    src = edge_index[0]
    a, b = _node_transform(
        node_feature, W1[:D], W1[D:], b1.reshape(1, H)
    )
    a_pad = jnp.pad(a, ((0, NP - N), (0, 0)))
    src_pad = jnp.pad(src, (0, (NP - N) * DEG))
    w2 = W2.reshape(H)
    b2v = jnp.broadcast_to(b2.reshape(1), (16,)).astype(jnp.float32)
    out = _edge_call(a_pad, b, src_pad, w2, b2v)
    return out[:N]


# speedup vs baseline: 1.0817x; 1.0817x over previous
import functools

import jax
import jax.numpy as jnp
from jax import lax
from jax.experimental import pallas as pl
from jax.experimental.pallas import tpu as pltpu
from jax.experimental.pallas import tpu_sc as plsc

N = 10000
D = 128
H = 128
DEG = 32

NC = 2
NS = 16
NW = NC * NS

CHUNK_D = 4
CHUNK_E = CHUNK_D * DEG
TRIPS = 80
NP = NW * TRIPS * CHUNK_D


def _mm_body(x_ref, w1a_ref, w1b_ref, b1_ref, a_ref, b_ref):
    x = x_ref[...]
    a_ref[...] = (
        jnp.dot(x, w1a_ref[...], preferred_element_type=jnp.float32)
        + b1_ref[...]
    )
    b_ref[...] = jnp.dot(x, w1b_ref[...], preferred_element_type=jnp.float32)


def _node_transform(x, w1a, w1b, b1):
    blk = 2000
    return pl.pallas_call(
        _mm_body,
        grid=(N // blk,),
        in_specs=[
            pl.BlockSpec((blk, D), lambda i: (i, 0)),
            pl.BlockSpec((D, H), lambda i: (0, 0)),
            pl.BlockSpec((D, H), lambda i: (0, 0)),
            pl.BlockSpec((1, H), lambda i: (0, 0)),
        ],
        out_specs=[
            pl.BlockSpec((blk, H), lambda i: (i, 0)),
            pl.BlockSpec((blk, H), lambda i: (i, 0)),
        ],
        out_shape=[
            jax.ShapeDtypeStruct((N, H), jnp.float32),
            jax.ShapeDtypeStruct((N, H), jnp.float32),
        ],
    )(x, w1a, w1b, b1)


_DNUMS = lax.GatherDimensionNumbers(
    offset_dims=(), collapsed_slice_dims=(0,), start_index_map=(0,)
)


def _perm(v, idx):
    return lax.gather(
        v, idx[:, None], _DNUMS, (1,),
        mode=lax.GatherScatterMode.PROMISE_IN_BOUNDS,
    )


def _edge_body(a_hbm, b_hbm, src_hbm, w2_hbm, b2_hbm, out_hbm,
               idx_v, rows_v, a_v, out_v, w2_v, b2_v, sem0, sem1):
    cid = lax.axis_index("c")
    sid = lax.axis_index("s")
    wid = sid * NC + cid

    pltpu.sync_copy(w2_hbm, w2_v)
    pltpu.sync_copy(b2_hbm, b2_v)

    b2vec = b2_v[...]
    iota16 = lax.iota(jnp.int32, 16)
    w_chunks = [w2_v[pl.ds(16 * jb, 16)] for jb in range(H // 16)]
    strides = (1, 2, 4, 8)
    perm_idx = [iota16 ^ s for s in strides]
    masks = [(iota16 & s) == 0 for s in strides]
    sems = (sem0, sem1)

    def base_of(t):
        return (t * NW + wid) * CHUNK_D

    def load_meta(t, p):
        base_d = base_of(t)
        pltpu.sync_copy(src_hbm.at[pl.ds(base_d * DEG, CHUNK_E)],
                        idx_v.at[p])
        pltpu.sync_copy(a_hbm.at[pl.ds(base_d, CHUNK_D)], a_v.at[p])

    def gather_start(p):
        pltpu.async_copy(b_hbm.at[idx_v.at[p]], rows_v.at[p], sems[p])

    def gather_wait(p):
        pltpu.make_async_copy(
            b_hbm.at[idx_v.at[p]], rows_v.at[p], sems[p]
        ).wait()

    def compute(t, p):
        def half_body(h, _):
            d = h // 2
            a_chunks = [
                a_v[p, d, pl.ds(16 * jb, 16)] for jb in range(H // 16)
            ]
            level = []
            for e in range(16):
                row = h * 16 + e
                acc = None
                for jb in range(H // 16):
                    g = rows_v[p, row, pl.ds(16 * jb, 16)]
                    t_ = g + a_chunks[jb]
                    u = jnp.maximum(t_, t_ * 0.01)
                    m = u * w_chunks[jb]
                    acc = m if acc is None else acc + m
                level.append(acc)
            for li in range(4):
                nxt = []
                for q in range(0, len(level), 2):
                    va, vb = level[q], level[q + 1]
                    hi = jnp.where(masks[li], va, vb)
                    lo = jnp.where(masks[li], vb, va)
                    nxt.append(hi + _perm(lo, perm_idx[li]))
                level = nxt
            out_v[d, pl.ds((h % 2) * 16, 16)] = level[0] + b2vec
            return 0

        lax.fori_loop(0, CHUNK_D * 2, half_body, 0)
        pltpu.sync_copy(out_v, out_hbm.at[pl.ds(base_of(t), CHUNK_D)])

    def chunk_body(t, _):
        load_meta(t, 0)
        gather_start(0)
        gather_wait(0)
        compute(t, 0)
        return 0

    lax.fori_loop(0, TRIPS, chunk_body, 0)


_edge_call = functools.partial(
    pl.kernel,
    mesh=plsc.VectorSubcoreMesh(core_axis_name="c", subcore_axis_name="s"),
    out_type=jax.ShapeDtypeStruct((NP, DEG), jnp.float32),
    scratch_types=[
        pltpu.VMEM((2, CHUNK_E), jnp.int32),
        pltpu.VMEM((2, CHUNK_E, H), jnp.float32),
        pltpu.VMEM((2, CHUNK_D, H), jnp.float32),
        pltpu.VMEM((CHUNK_D, DEG), jnp.float32),
        pltpu.VMEM((H,), jnp.float32),
        pltpu.VMEM((16,), jnp.float32),
        pltpu.SemaphoreType.DMA,
        pltpu.SemaphoreType.DMA,
    ],
)(_edge_body)


def kernel(node_feature, edge_index, W1, b1, W2, b2):
    src = edge_index[0]
    a, b = _node_transform(
        node_feature, W1[:D], W1[D:], b1.reshape(1, H)
    )
    a_pad = jnp.pad(a, ((0, NP - N), (0, 0)))
    src_pad = jnp.pad(src, (0, (NP - N) * DEG))
    w2 = W2.reshape(H)
    b2v = jnp.broadcast_to(b2.reshape(1), (16,)).astype(jnp.float32)
    out = _edge_call(a_pad, b, src_pad, w2, b2v)
    return out[:N]


# speedup vs baseline: 1.0830x; 1.0012x over previous
import functools

import jax
import jax.numpy as jnp
from jax import lax
from jax.experimental import pallas as pl
from jax.experimental.pallas import tpu as pltpu
from jax.experimental.pallas import tpu_sc as plsc

N = 10000
D = 128
H = 128
DEG = 32

NC = 2
NS = 16
NW = NC * NS

CHUNK_D = 4
CHUNK_E = CHUNK_D * DEG
TRIPS = 80
NP = NW * TRIPS * CHUNK_D


def _mm_body(x_ref, w1a_ref, w1b_ref, b1_ref, a_ref, b_ref):
    x = x_ref[...]
    a_ref[...] = (
        jnp.dot(x, w1a_ref[...], preferred_element_type=jnp.float32)
        + b1_ref[...]
    )
    b_ref[...] = jnp.dot(x, w1b_ref[...], preferred_element_type=jnp.float32)


def _node_transform(x, w1a, w1b, b1):
    blk = 2000
    return pl.pallas_call(
        _mm_body,
        grid=(N // blk,),
        in_specs=[
            pl.BlockSpec((blk, D), lambda i: (i, 0)),
            pl.BlockSpec((D, H), lambda i: (0, 0)),
            pl.BlockSpec((D, H), lambda i: (0, 0)),
            pl.BlockSpec((1, H), lambda i: (0, 0)),
        ],
        out_specs=[
            pl.BlockSpec((blk, H), lambda i: (i, 0)),
            pl.BlockSpec((blk, H), lambda i: (i, 0)),
        ],
        out_shape=[
            jax.ShapeDtypeStruct((N, H), jnp.float32),
            jax.ShapeDtypeStruct((N, H), jnp.float32),
        ],
    )(x, w1a, w1b, b1)


_DNUMS = lax.GatherDimensionNumbers(
    offset_dims=(), collapsed_slice_dims=(0,), start_index_map=(0,)
)


def _perm(v, idx):
    return lax.gather(
        v, idx[:, None], _DNUMS, (1,),
        mode=lax.GatherScatterMode.PROMISE_IN_BOUNDS,
    )


def _edge_body(a_hbm, b_hbm, src_hbm, w2_hbm, b2_hbm, out_hbm,
               idx_v, rows_v, a_v, out_v, w2_v, b2_v, sem0, sem1):
    cid = lax.axis_index("c")
    sid = lax.axis_index("s")
    wid = sid * NC + cid

    pltpu.sync_copy(w2_hbm, w2_v)
    pltpu.sync_copy(b2_hbm, b2_v)

    b2vec = b2_v[...]
    iota16 = lax.iota(jnp.int32, 16)
    w_chunks = [w2_v[pl.ds(16 * jb, 16)] for jb in range(H // 16)]
    strides = (1, 2, 4, 8)
    perm_idx = [iota16 ^ s for s in strides]
    masks = [(iota16 & s) == 0 for s in strides]
    sems = (sem0, sem1)

    def base_of(t):
        return (t * NW + wid) * CHUNK_D

    def load_meta(t, p):
        base_d = base_of(t)
        pltpu.sync_copy(src_hbm.at[pl.ds(base_d * DEG, CHUNK_E)],
                        idx_v.at[p])
        pltpu.sync_copy(a_hbm.at[pl.ds(base_d, CHUNK_D)], a_v.at[p])

    def gather_start(p):
        pltpu.async_copy(b_hbm.at[idx_v.at[p]], rows_v.at[p], sems[p])

    def gather_wait(p):
        pltpu.make_async_copy(
            b_hbm.at[idx_v.at[p]], rows_v.at[p], sems[p]
        ).wait()

    def compute(t, p):
        @plsc.parallel_loop(0, CHUNK_D)
        def dst_body(d):
            a_chunks = [
                a_v[p, d, pl.ds(16 * jb, 16)] for jb in range(H // 16)
            ]
            for half in range(DEG // 16):
                level = []
                for e in range(16):
                    row = d * DEG + half * 16 + e
                    acc = None
                    for jb in range(H // 16):
                        g = rows_v[p, row, pl.ds(16 * jb, 16)]
                        t_ = g + a_chunks[jb]
                        u = jnp.maximum(t_, t_ * 0.01)
                        m = u * w_chunks[jb]
                        acc = m if acc is None else acc + m
                    level.append(acc)
                for li in range(4):
                    nxt = []
                    for q in range(0, len(level), 2):
                        va, vb = level[q], level[q + 1]
                        hi = jnp.where(masks[li], va, vb)
                        lo = jnp.where(masks[li], vb, va)
                        nxt.append(hi + _perm(lo, perm_idx[li]))
                    level = nxt
                out_v[d, pl.ds(half * 16, 16)] = level[0] + b2vec

        pltpu.sync_copy(out_v, out_hbm.at[pl.ds(base_of(t), CHUNK_D)])

    def chunk_body(t, _):
        load_meta(t, 0)
        gather_start(0)
        gather_wait(0)
        compute(t, 0)
        return 0

    lax.fori_loop(0, TRIPS, chunk_body, 0)


_edge_call = functools.partial(
    pl.kernel,
    mesh=plsc.VectorSubcoreMesh(core_axis_name="c", subcore_axis_name="s"),
    out_type=jax.ShapeDtypeStruct((NP, DEG), jnp.float32),
    scratch_types=[
        pltpu.VMEM((2, CHUNK_E), jnp.int32),
        pltpu.VMEM((2, CHUNK_E, H), jnp.float32),
        pltpu.VMEM((2, CHUNK_D, H), jnp.float32),
        pltpu.VMEM((CHUNK_D, DEG), jnp.float32),
        pltpu.VMEM((H,), jnp.float32),
        pltpu.VMEM((16,), jnp.float32),
        pltpu.SemaphoreType.DMA,
        pltpu.SemaphoreType.DMA,
    ],
)(_edge_body)


def kernel(node_feature, edge_index, W1, b1, W2, b2):
    src = edge_index[0]
    a, b = _node_transform(
        node_feature, W1[:D], W1[D:], b1.reshape(1, H)
    )
    a_pad = jnp.pad(a, ((0, NP - N), (0, 0)))
    src_pad = jnp.pad(src, (0, (NP - N) * DEG))
    w2 = W2.reshape(H)
    b2v = jnp.broadcast_to(b2.reshape(1), (16,)).astype(jnp.float32)
    out = _edge_call(a_pad, b, src_pad, w2, b2v)
    return out[:N]


# speedup vs baseline: 1.5573x; 1.4379x over previous
import functools

import jax
import jax.numpy as jnp
from jax import lax
from jax.experimental import pallas as pl
from jax.experimental.pallas import tpu as pltpu
from jax.experimental.pallas import tpu_sc as plsc

N = 10000
D = 128
H = 128
DEG = 32

NC = 2
NS = 16
NW = NC * NS

CHUNK_D = 4
CHUNK_E = CHUNK_D * DEG
TRIPS = 80
TEC_D = TRIPS * CHUNK_D
NP = NW * TEC_D


def _mm_body(x_ref, w1a_ref, w1b_ref, b1_ref, a_ref, b_ref):
    x = x_ref[...]
    a_ref[...] = (
        jnp.dot(x, w1a_ref[...], preferred_element_type=jnp.float32)
        + b1_ref[...]
    )
    b_ref[...] = jnp.dot(x, w1b_ref[...], preferred_element_type=jnp.float32)


def _node_transform(x, w1a, w1b, b1):
    blk = 2000
    return pl.pallas_call(
        _mm_body,
        grid=(N // blk,),
        in_specs=[
            pl.BlockSpec((blk, D), lambda i: (i, 0)),
            pl.BlockSpec((D, H), lambda i: (0, 0)),
            pl.BlockSpec((D, H), lambda i: (0, 0)),
            pl.BlockSpec((1, H), lambda i: (0, 0)),
        ],
        out_specs=[
            pl.BlockSpec((blk, H), lambda i: (i, 0)),
            pl.BlockSpec((blk, H), lambda i: (i, 0)),
        ],
        out_shape=[
            jax.ShapeDtypeStruct((N, H), jnp.float32),
            jax.ShapeDtypeStruct((N, H), jnp.float32),
        ],
    )(x, w1a, w1b, b1)


_DNUMS = lax.GatherDimensionNumbers(
    offset_dims=(), collapsed_slice_dims=(0,), start_index_map=(0,)
)


def _perm(v, idx):
    return lax.gather(
        v, idx[:, None], _DNUMS, (1,),
        mode=lax.GatherScatterMode.PROMISE_IN_BOUNDS,
    )


def _edge_body(a_hbm, b_hbm, src_hbm, w2_hbm, b2_hbm, out_hbm,
               idx_v, rows_v, a_v, out_v, w2_v, b2_v, sem0, sem1):
    cid = lax.axis_index("c")
    sid = lax.axis_index("s")
    wid = sid * NC + cid

    pltpu.sync_copy(w2_hbm, w2_v)
    pltpu.sync_copy(b2_hbm, b2_v)
    pltpu.sync_copy(src_hbm.at[pl.ds(wid * TRIPS, TRIPS)], idx_v)
    pltpu.sync_copy(a_hbm.at[pl.ds(wid * TEC_D, TEC_D)], a_v)

    b2vec = b2_v[...]
    iota16 = lax.iota(jnp.int32, 16)
    w_chunks = [w2_v[pl.ds(16 * jb, 16)] for jb in range(H // 16)]
    strides = (1, 2, 4, 8)
    perm_idx = [iota16 ^ s for s in strides]
    masks = [(iota16 & s) == 0 for s in strides]
    sems = (sem0, sem1)

    def gather_start(i, p):
        pltpu.async_copy(b_hbm.at[idx_v.at[i]], rows_v.at[p], sems[p])

    def gather_wait(i, p):
        pltpu.make_async_copy(
            b_hbm.at[idx_v.at[i]], rows_v.at[p], sems[p]
        ).wait()

    gather_start(0, 0)

    def compute(i, p):
        def dst_body(d, _):
            a_chunks = [
                a_v[i * CHUNK_D + d, pl.ds(16 * jb, 16)]
                for jb in range(H // 16)
            ]
            for half in range(DEG // 16):
                level = []
                for e in range(16):
                    row = d * DEG + half * 16 + e
                    acc = None
                    for jb in range(H // 16):
                        g = rows_v[p, row, pl.ds(16 * jb, 16)]
                        t = g + a_chunks[jb]
                        u = jnp.maximum(t, t * 0.01)
                        m = u * w_chunks[jb]
                        acc = m if acc is None else acc + m
                    level.append(acc)
                for li in range(4):
                    nxt = []
                    for q in range(0, len(level), 2):
                        va, vb = level[q], level[q + 1]
                        hi = jnp.where(masks[li], va, vb)
                        lo = jnp.where(masks[li], vb, va)
                        nxt.append(hi + _perm(lo, perm_idx[li]))
                    level = nxt
                out_v[i * CHUNK_D + d, pl.ds(half * 16, 16)] = (
                    level[0] + b2vec
                )
            return 0

        lax.fori_loop(0, CHUNK_D, dst_body, 0)

    def pair_body(k, _):
        i0 = 2 * k
        gather_start(i0 + 1, 1)
        gather_wait(i0, 0)
        compute(i0, 0)
        gather_start(i0 + 2, 0)
        gather_wait(i0 + 1, 1)
        compute(i0 + 1, 1)
        return 0

    lax.fori_loop(0, TRIPS // 2 - 1, pair_body, 0)
    gather_start(TRIPS - 1, 1)
    gather_wait(TRIPS - 2, 0)
    compute(TRIPS - 2, 0)
    gather_wait(TRIPS - 1, 1)
    compute(TRIPS - 1, 1)

    pltpu.sync_copy(out_v, out_hbm.at[pl.ds(wid * TEC_D, TEC_D)])


_edge_call = functools.partial(
    pl.kernel,
    mesh=plsc.VectorSubcoreMesh(core_axis_name="c", subcore_axis_name="s"),
    out_type=jax.ShapeDtypeStruct((NP, DEG), jnp.float32),
    scratch_types=[
        pltpu.VMEM((TRIPS, CHUNK_E), jnp.int32),
        pltpu.VMEM((2, CHUNK_E, H), jnp.float32),
        pltpu.VMEM((TEC_D, H), jnp.float32),
        pltpu.VMEM((TEC_D, DEG), jnp.float32),
        pltpu.VMEM((H,), jnp.float32),
        pltpu.VMEM((16,), jnp.float32),
        pltpu.SemaphoreType.DMA,
        pltpu.SemaphoreType.DMA,
    ],
)(_edge_body)


def kernel(node_feature, edge_index, W1, b1, W2, b2):
    src = edge_index[0]
    a, b = _node_transform(
        node_feature, W1[:D], W1[D:], b1.reshape(1, H)
    )
    a_pad = jnp.pad(a, ((0, NP - N), (0, 0)))
    src_pad = jnp.pad(src, (0, (NP - N) * DEG)).reshape(NW * TRIPS, CHUNK_E)
    w2 = W2.reshape(H)
    b2v = jnp.broadcast_to(b2.reshape(1), (16,)).astype(jnp.float32)
    out = _edge_call(a_pad, b, src_pad, w2, b2v)
    return out[:N]


# speedup vs baseline: 3.1638x; 2.0316x over previous
import functools

import jax
import jax.numpy as jnp
from jax import lax
from jax.experimental import pallas as pl
from jax.experimental.pallas import tpu as pltpu
from jax.experimental.pallas import tpu_sc as plsc

N = 10000
D = 128
H = 128
DEG = 32

NC = 2
NS = 16
NW = NC * NS

CHUNK_D = 4
CHUNK_E = CHUNK_D * DEG
TRIPS = 80
TEC_D = TRIPS * CHUNK_D
NP = NW * TEC_D


def _mm_body(x_ref, w1a_ref, w1b_ref, b1_ref, a_ref, b_ref):
    x = x_ref[...]
    a_ref[...] = (
        jnp.dot(x, w1a_ref[...], preferred_element_type=jnp.float32)
        + b1_ref[...]
    )
    b_ref[...] = jnp.dot(x, w1b_ref[...], preferred_element_type=jnp.float32)


def _node_transform(x, w1a, w1b, b1):
    blk = 2000
    return pl.pallas_call(
        _mm_body,
        grid=(N // blk,),
        in_specs=[
            pl.BlockSpec((blk, D), lambda i: (i, 0)),
            pl.BlockSpec((D, H), lambda i: (0, 0)),
            pl.BlockSpec((D, H), lambda i: (0, 0)),
            pl.BlockSpec((1, H), lambda i: (0, 0)),
        ],
        out_specs=[
            pl.BlockSpec((blk, H), lambda i: (i, 0)),
            pl.BlockSpec((blk, H), lambda i: (i, 0)),
        ],
        out_shape=[
            jax.ShapeDtypeStruct((N, H), jnp.float32),
            jax.ShapeDtypeStruct((N, H), jnp.float32),
        ],
    )(x, w1a, w1b, b1)


_DNUMS = lax.GatherDimensionNumbers(
    offset_dims=(), collapsed_slice_dims=(0,), start_index_map=(0,)
)


def _perm(v, idx):
    return lax.gather(
        v, idx[:, None], _DNUMS, (1,),
        mode=lax.GatherScatterMode.PROMISE_IN_BOUNDS,
    )


def _edge_body(a_hbm, b_hbm, src_hbm, w2_hbm, b2_hbm, out_hbm,
               idx_v, rows_v, a_v, out_v, w2_v, b2_v, b_sh, sem0, sem1):
    cid = lax.axis_index("c")
    sid = lax.axis_index("s")
    wid = sid * NC + cid

    stripe = NP // NS
    pltpu.sync_copy(b_hbm.at[pl.ds(sid * stripe, stripe)],
                    b_sh.at[pl.ds(sid * stripe, stripe)])

    pltpu.sync_copy(w2_hbm, w2_v)
    pltpu.sync_copy(b2_hbm, b2_v)
    pltpu.sync_copy(src_hbm.at[pl.ds(wid * TRIPS, TRIPS)], idx_v)
    plsc.subcore_barrier()

    b2vec = b2_v[...]
    iota16 = lax.iota(jnp.int32, 16)
    w_chunks = [w2_v[pl.ds(16 * jb, 16)] for jb in range(H // 16)]
    strides = (1, 2, 4, 8)
    perm_idx = [iota16 ^ s for s in strides]
    masks = [(iota16 & s) == 0 for s in strides]
    sems = (sem0, sem1)

    def gather_start(i, p):
        pltpu.async_copy(b_sh.at[idx_v.at[i]], rows_v.at[p], sems[p])

    def gather_wait(i, p):
        pltpu.make_async_copy(
            b_sh.at[idx_v.at[i]], rows_v.at[p], sems[p]
        ).wait()

    def copy_a(i, p):
        pltpu.sync_copy(
            a_hbm.at[pl.ds(wid * TEC_D + i * CHUNK_D, CHUNK_D)], a_v.at[p]
        )

    copy_a(0, 0)
    gather_start(0, 0)

    def compute(i, p):
        def dst_body(d, _):
            a_chunks = [
                a_v[p, d, pl.ds(16 * jb, 16)] for jb in range(H // 16)
            ]
            for half in range(DEG // 16):
                level = []
                for e in range(16):
                    row = d * DEG + half * 16 + e
                    acc = None
                    for jb in range(H // 16):
                        g = rows_v[p, row, pl.ds(16 * jb, 16)]
                        t = g + a_chunks[jb]
                        u = jnp.maximum(t, t * 0.01)
                        m = u * w_chunks[jb]
                        acc = m if acc is None else acc + m
                    level.append(acc)
                for li in range(4):
                    nxt = []
                    for q in range(0, len(level), 2):
                        va, vb = level[q], level[q + 1]
                        hi = jnp.where(masks[li], va, vb)
                        lo = jnp.where(masks[li], vb, va)
                        nxt.append(hi + _perm(lo, perm_idx[li]))
                    level = nxt
                out_v[d, pl.ds(half * 16, 16)] = level[0] + b2vec
            return 0

        lax.fori_loop(0, CHUNK_D, dst_body, 0)
        pltpu.sync_copy(
            out_v, out_hbm.at[pl.ds(wid * TEC_D + i * CHUNK_D, CHUNK_D)]
        )

    def pair_body(k, _):
        i0 = 2 * k
        gather_start(i0 + 1, 1)
        copy_a(i0 + 1, 1)
        gather_wait(i0, 0)
        compute(i0, 0)
        gather_start(i0 + 2, 0)
        copy_a(i0 + 2, 0)
        gather_wait(i0 + 1, 1)
        compute(i0 + 1, 1)
        return 0

    lax.fori_loop(0, TRIPS // 2 - 1, pair_body, 0)
    gather_start(TRIPS - 1, 1)
    copy_a(TRIPS - 1, 1)
    gather_wait(TRIPS - 2, 0)
    compute(TRIPS - 2, 0)
    gather_wait(TRIPS - 1, 1)
    compute(TRIPS - 1, 1)


_edge_call = functools.partial(
    pl.kernel,
    mesh=plsc.VectorSubcoreMesh(core_axis_name="c", subcore_axis_name="s"),
    out_type=jax.ShapeDtypeStruct((NP, DEG), jnp.float32),
    scratch_types=[
        pltpu.VMEM((TRIPS, CHUNK_E), jnp.int32),
        pltpu.VMEM((2, CHUNK_E, H), jnp.float32),
        pltpu.VMEM((2, CHUNK_D, H), jnp.float32),
        pltpu.VMEM((CHUNK_D, DEG), jnp.float32),
        pltpu.VMEM((H,), jnp.float32),
        pltpu.VMEM((16,), jnp.float32),
        pltpu.VMEM_SHARED((NP, H), jnp.float32),
        pltpu.SemaphoreType.DMA,
        pltpu.SemaphoreType.DMA,
    ],
)(_edge_body)


def kernel(node_feature, edge_index, W1, b1, W2, b2):
    src = edge_index[0]
    a, b = _node_transform(
        node_feature, W1[:D], W1[D:], b1.reshape(1, H)
    )
    a_pad = jnp.pad(a, ((0, NP - N), (0, 0)))
    b_pad = jnp.pad(b, ((0, NP - N), (0, 0)))
    src_pad = jnp.pad(src, (0, (NP - N) * DEG)).reshape(NW * TRIPS, CHUNK_E)
    w2 = W2.reshape(H)
    b2v = jnp.broadcast_to(b2.reshape(1), (16,)).astype(jnp.float32)
    out = _edge_call(a_pad, b_pad, src_pad, w2, b2v)
    return out[:N]


# speedup vs baseline: 3.1950x; 1.0099x over previous
import functools

import jax
import jax.numpy as jnp
from jax import lax
from jax.experimental import pallas as pl
from jax.experimental.pallas import tpu as pltpu
from jax.experimental.pallas import tpu_sc as plsc

N = 10000
D = 128
H = 128
DEG = 32

NC = 2
NS = 16
NW = NC * NS

CHUNK_D = 4
CHUNK_E = CHUNK_D * DEG
TRIPS = 80
TEC_D = TRIPS * CHUNK_D
NP = NW * TEC_D


def _mm_body(x_ref, w1a_ref, w1b_ref, b1_ref, a_ref, b_ref):
    x = x_ref[...]
    a_ref[...] = (
        jnp.dot(x, w1a_ref[...], preferred_element_type=jnp.float32)
        + b1_ref[...]
    )
    b_ref[...] = jnp.dot(x, w1b_ref[...], preferred_element_type=jnp.float32)


def _node_transform(x, w1a, w1b, b1):
    blk = 2000
    return pl.pallas_call(
        _mm_body,
        grid=(N // blk,),
        in_specs=[
            pl.BlockSpec((blk, D), lambda i: (i, 0)),
            pl.BlockSpec((D, H), lambda i: (0, 0)),
            pl.BlockSpec((D, H), lambda i: (0, 0)),
            pl.BlockSpec((1, H), lambda i: (0, 0)),
        ],
        out_specs=[
            pl.BlockSpec((blk, H), lambda i: (i, 0)),
            pl.BlockSpec((blk, H), lambda i: (i, 0)),
        ],
        out_shape=[
            jax.ShapeDtypeStruct((N, H), jnp.float32),
            jax.ShapeDtypeStruct((N, H), jnp.float32),
        ],
    )(x, w1a, w1b, b1)


_DNUMS = lax.GatherDimensionNumbers(
    offset_dims=(), collapsed_slice_dims=(0,), start_index_map=(0,)
)


def _perm(v, idx):
    return lax.gather(
        v, idx[:, None], _DNUMS, (1,),
        mode=lax.GatherScatterMode.PROMISE_IN_BOUNDS,
    )


def _edge_body(a_hbm, b_hbm, src_hbm, w2_hbm, b2_hbm, out_hbm,
               idx_v, rows_v, a_v, out_v, w2_v, b2_v, b_sh, sem0, sem1):
    cid = lax.axis_index("c")
    sid = lax.axis_index("s")
    wid = sid * NC + cid

    stripe = NP // NS
    pltpu.sync_copy(b_hbm.at[pl.ds(sid * stripe, stripe)],
                    b_sh.at[pl.ds(sid * stripe, stripe)])

    pltpu.sync_copy(w2_hbm, w2_v)
    pltpu.sync_copy(b2_hbm, b2_v)
    pltpu.sync_copy(src_hbm.at[pl.ds(wid * TRIPS, TRIPS)], idx_v)
    plsc.subcore_barrier()

    b2vec = b2_v[...]
    iota16 = lax.iota(jnp.int32, 16)
    w_chunks = [w2_v[pl.ds(16 * jb, 16)] for jb in range(H // 16)]
    strides = (1, 2, 4, 8)
    perm_idx = [iota16 ^ s for s in strides]
    masks = [(iota16 & s) == 0 for s in strides]
    sems = (sem0, sem1)

    def gather_start(i, p):
        pltpu.async_copy(b_sh.at[idx_v.at[i]], rows_v.at[p], sems[p])

    def gather_wait(i, p):
        pltpu.make_async_copy(
            b_sh.at[idx_v.at[i]], rows_v.at[p], sems[p]
        ).wait()

    def copy_a(i, p):
        pltpu.sync_copy(
            a_hbm.at[pl.ds(wid * TEC_D + i * CHUNK_D, CHUNK_D)], a_v.at[p]
        )

    copy_a(0, 0)
    gather_start(0, 0)

    def compute(i, p):
        def dst_body(d, _):
            a_chunks = [
                a_v[p, d, pl.ds(16 * jb, 16)] for jb in range(H // 16)
            ]
            for half in range(DEG // 16):
                level = []
                for e in range(16):
                    row = d * DEG + half * 16 + e
                    acc = None
                    for jb in range(H // 16):
                        g = rows_v[p, row, pl.ds(16 * jb, 16)]
                        t = g + a_chunks[jb]
                        u = jnp.maximum(t, t * 0.01)
                        m = u * w_chunks[jb]
                        acc = m if acc is None else acc + m
                    level.append(acc)
                for li in range(4):
                    nxt = []
                    for q in range(0, len(level), 2):
                        va, vb = level[q], level[q + 1]
                        hi = jnp.where(masks[li], va, vb)
                        lo = jnp.where(masks[li], vb, va)
                        nxt.append(hi + _perm(lo, perm_idx[li]))
                    level = nxt
                out_v[d, pl.ds(half * 16, 16)] = level[0] + b2vec
            return 0

        lax.fori_loop(0, CHUNK_D, dst_body, 0)
        pltpu.sync_copy(
            out_v, out_hbm.at[pl.ds(wid * TEC_D + i * CHUNK_D, CHUNK_D)]
        )

    def chunk_loop(i, _):
        gather_start(i, 0)
        copy_a(i, 0)
        gather_wait(i, 0)
        compute(i, 0)
        return 0

    lax.fori_loop(0, TRIPS, chunk_loop, 0)


_edge_call = functools.partial(
    pl.kernel,
    mesh=plsc.VectorSubcoreMesh(core_axis_name="c", subcore_axis_name="s"),
    out_type=jax.ShapeDtypeStruct((NP, DEG), jnp.float32),
    scratch_types=[
        pltpu.VMEM((TRIPS, CHUNK_E), jnp.int32),
        pltpu.VMEM((2, CHUNK_E, H), jnp.float32),
        pltpu.VMEM((2, CHUNK_D, H), jnp.float32),
        pltpu.VMEM((CHUNK_D, DEG), jnp.float32),
        pltpu.VMEM((H,), jnp.float32),
        pltpu.VMEM((16,), jnp.float32),
        pltpu.VMEM_SHARED((NP, H), jnp.float32),
        pltpu.SemaphoreType.DMA,
        pltpu.SemaphoreType.DMA,
    ],
)(_edge_body)


def kernel(node_feature, edge_index, W1, b1, W2, b2):
    src = edge_index[0]
    a, b = _node_transform(
        node_feature, W1[:D], W1[D:], b1.reshape(1, H)
    )
    a_pad = jnp.pad(a, ((0, NP - N), (0, 0)))
    b_pad = jnp.pad(b, ((0, NP - N), (0, 0)))
    src_pad = jnp.pad(src, (0, (NP - N) * DEG)).reshape(NW * TRIPS, CHUNK_E)
    w2 = W2.reshape(H)
    b2v = jnp.broadcast_to(b2.reshape(1), (16,)).astype(jnp.float32)
    out = _edge_call(a_pad, b_pad, src_pad, w2, b2v)
    return out[:N]
